# baseline (speedup 1.0000x reference)
import functools

import jax
import jax.numpy as jnp
from jax import lax
from jax.experimental import pallas as pl
from jax.experimental.pallas import tpu as pltpu
from jax.experimental.pallas import tpu_sc as plsc

B = 16
C = 4
H = 512
W = 512
HW = H * W
NSEG = 1024
NC = 2
NS = 16
L = 16
HROWS = H // 2

RPC = 4
CHUNK = RPC * W
NCHUNK = HROWS // RPC
VPR = W // L
NF = 5
LANE_TAB = NSEG * NF
TAB = L * LANE_TAB

_mesh = plsc.VectorSubcoreMesh(
    core_axis_name="c", subcore_axis_name="s", num_cores=NC, num_subcores=NS
)
_params = pltpu.CompilerParams(needs_layout_passes=False)


def _iota16():
    return lax.iota(jnp.int32, L)


@functools.partial(
    pl.kernel,
    out_type=jax.ShapeDtypeStruct((NC * NS, L), jnp.float32),
    mesh=_mesh,
    compiler_params=_params,
    scratch_types=[
        pltpu.VMEM((TAB,), jnp.float32),
        pltpu.VMEM((LANE_TAB,), jnp.float32),
        pltpu.VMEM((LANE_TAB,), jnp.float32),
        pltpu.VMEM((2 * RPC, W), jnp.int32),
        pltpu.VMEM((2 * RPC, W), jnp.int32),
        pltpu.VMEM((2 * C * RPC, W), jnp.float32),
        pltpu.VMEM((2 * NSEG,), jnp.int32),
        pltpu.VMEM((L,), jnp.float32),
        pltpu.VMEM((L,), jnp.float32),
        pltpu.VMEM_SHARED((NS, LANE_TAB), jnp.float32),
        pltpu.SemaphoreType.DMA,
        pltpu.SemaphoreType.DMA,
    ],
)
def _superpixel(is_hbm, lbl_hbm, il_hbm, th_hbm, out_hbm,
                tab, comb, part, lblb, ilb, chb, pckb, thb, accb, shr,
                sem0, sem1):
    s = lax.axis_index("s")
    c = lax.axis_index("c")
    b = c * (B // NC) + s // 2
    half = s % 2
    row0 = half * HROWS
    orow = c * NS + s
    sems = (sem0, sem1)

    def _copies1(g, p, sem):
        r0 = row0 + g * RPC
        cps = [pltpu.make_async_copy(
            lbl_hbm.at[b, 0, pl.ds(r0, RPC), :],
            lblb.at[pl.ds(p * RPC, RPC), :], sem)]
        for ch in range(C):
            cps.append(pltpu.make_async_copy(
                is_hbm.at[b, ch, pl.ds(r0, RPC), :],
                chb.at[pl.ds((p * C + ch) * RPC, RPC), :], sem))
        return cps

    def _copies2(g, p, sem):
        return _copies1(g, p, sem) + [pltpu.make_async_copy(
            il_hbm.at[b, pl.ds(row0 + g * RPC, RPC), :],
            ilb.at[pl.ds(p * RPC, RPC), :], sem)]

    zero = jnp.zeros((L,), jnp.float32)

    pltpu.sync_copy(th_hbm, thb)
    tv = thb[...]

    @plsc.parallel_loop(0, TAB // L, unroll=8)
    def _z(j):
        tab[pl.ds(j * L, L)] = zero

    iotav = _iota16()
    ones = jnp.full((L,), 1.0, jnp.float32)

    for p in range(2):
        for cp in _copies1(p, p, sems[p]):
            cp.start()

    @pl.loop(0, NCHUNK, step=2)
    def _pair1(g):
        for p in range(2):
            gg = g + p
            for cp in _copies1(gg, p, sems[p]):
                cp.wait()

            @plsc.parallel_loop(0, VPR, unroll=4)
            def _vreg(k):
                for r in range(RPC):
                    lbl = lblb[p * RPC + r, pl.ds(k * L, L)]
                    idx0 = (lbl << 4) + iotav
                    for ch in range(C):
                        v = chb[(p * C + ch) * RPC + r, pl.ds(k * L, L)]
                        plsc.addupdate_scatter(
                            tab, [idx0 + ch * (NSEG * L)], v)
                    plsc.addupdate_scatter(tab, [idx0 + C * (NSEG * L)],
                                           ones)

            @pl.when(gg + 2 < NCHUNK)
            def _():
                for cp in _copies1(gg + 2, p, sems[p]):
                    cp.start()

    last = iotav == (L - 1)

    @plsc.parallel_loop(0, LANE_TAB, unroll=4)
    def _red(e):
        cs = plsc.cumsum(tab[pl.ds(e * L, L)])
        plsc.store_scatter(comb, [iotav + (e - (L - 1))], cs, mask=last)

    pltpu.sync_copy(comb, shr.at[s])
    plsc.subcore_barrier()
    pltpu.sync_copy(shr.at[s ^ 1], part)

    @plsc.parallel_loop(0, LANE_TAB // L, unroll=2)
    def _add(j):
        o = j * L
        comb[pl.ds(o, L)] = comb[pl.ds(o, L)] + part[pl.ds(o, L)]

    def _bf16(a):
        u = plsc.bitcast(a, jnp.uint32)
        return (u + jnp.uint32(0x7FFF) + ((u >> 16) & jnp.uint32(1))) >> 16

    @plsc.parallel_loop(0, NSEG // L, unroll=2)
    def _avg(v):
        g0 = v * L
        n = comb[pl.ds(C * NSEG + g0, L)]
        inv = 1.0 / jnp.maximum(n, 1.0)
        keep = (g0 + _iota16()) != 0
        r = []
        for ch in range(C):
            a = comb[pl.ds(ch * NSEG + g0, L)] * inv
            a = jnp.where(keep, a, 0.0)
            r.append(_bf16(a))
        pckb[pl.ds(g0, L)] = plsc.bitcast((r[0] << 16) | r[1], jnp.int32)
        pckb[pl.ds(NSEG + g0, L)] = plsc.bitcast((r[2] << 16) | r[3],
                                                 jnp.int32)

    for p in range(2):
        for cp in _copies2(p, p, sems[p]):
            cp.start()

    def _pair2(g, acc):
        for p in range(2):
            gg = g * 2 + p
            for cp in _copies2(gg, p, sems[p]):
                cp.wait()

            @plsc.parallel_loop(0, VPR, unroll=4, carry=acc)
            def _vreg(k, a):
                hi = jnp.uint32(0xFFFF0000)
                for r in range(RPC):
                    lbl = lblb[p * RPC + r, pl.ds(k * L, L)]
                    il = ilb[p * RPC + r, pl.ds(k * L, L)]
                    q0 = plsc.bitcast(
                        plsc.load_gather(pckb, [lbl]), jnp.uint32)
                    q1 = plsc.bitcast(
                        plsc.load_gather(pckb, [lbl + NSEG]), jnp.uint32)
                    av = (plsc.bitcast(q0 & hi, jnp.float32),
                          plsc.bitcast(q0 << 16, jnp.float32),
                          plsc.bitcast(q1 & hi, jnp.float32),
                          plsc.bitcast(q1 << 16, jnp.float32))
                    nrm = zero
                    for ch in range(C):
                        v = chb[(p * C + ch) * RPC + r, pl.ds(k * L, L)]
                        d = v - av[ch]
                        nrm = nrm + d * d
                    w = jnp.where(il.astype(jnp.float32) > tv, 1.0, 0.0)
                    a = a + w * nrm
                return a
            acc = _vreg

            @pl.when(gg + 2 < NCHUNK)
            def _():
                for cp in _copies2(gg + 2, p, sems[p]):
                    cp.start()
        return acc

    acc = lax.fori_loop(0, NCHUNK // 2, _pair2, zero)
    accb[...] = acc
    pltpu.sync_copy(accb, out_hbm.at[orow])


def kernel(Is, Ispp, Il, line_thresh):
    th = jnp.full((L,), line_thresh, jnp.float32)
    parts = _superpixel(Is, Ispp, Il, th)
    return jnp.sum(parts) / (B * HW)


# speedup vs baseline: 1.2712x; 1.2712x over previous
import functools

import jax
import jax.numpy as jnp
from jax import lax
from jax.experimental import pallas as pl
from jax.experimental.pallas import tpu as pltpu
from jax.experimental.pallas import tpu_sc as plsc

B = 16
C = 4
H = 512
W = 512
HW = H * W
NSEG = 1024
NC = 2
NS = 16
L = 16
HROWS = H // 2

RPC = 4
CHUNK = RPC * W
NCHUNK = HROWS // RPC
VPR = W // L
NF = 5
LANE_TAB = NSEG * NF
TAB = L * LANE_TAB

_mesh = plsc.VectorSubcoreMesh(
    core_axis_name="c", subcore_axis_name="s", num_cores=NC, num_subcores=NS
)
_params = pltpu.CompilerParams(needs_layout_passes=False)


def _iota16():
    return lax.iota(jnp.int32, L)


@functools.partial(
    pl.kernel,
    out_type=jax.ShapeDtypeStruct((NC * NS, L), jnp.float32),
    mesh=_mesh,
    compiler_params=_params,
    scratch_types=[
        pltpu.VMEM((TAB,), jnp.float32),
        pltpu.VMEM((LANE_TAB,), jnp.float32),
        pltpu.VMEM((LANE_TAB,), jnp.float32),
        pltpu.VMEM((2 * RPC, W), jnp.int32),
        pltpu.VMEM((2 * RPC, W), jnp.int32),
        pltpu.VMEM((2 * C * RPC, W), jnp.float32),
        pltpu.VMEM((2 * NSEG,), jnp.int32),
        pltpu.VMEM((L,), jnp.float32),
        pltpu.VMEM((L,), jnp.float32),
        pltpu.VMEM_SHARED((NS, LANE_TAB), jnp.float32),
        pltpu.SemaphoreType.DMA,
        pltpu.SemaphoreType.DMA,
    ],
)
def _superpixel(is_hbm, lbl_hbm, il_hbm, th_hbm, out_hbm,
                tab, comb, part, lblb, ilb, chb, pckb, thb, accb, shr,
                sem0, sem1):
    s = lax.axis_index("s")
    c = lax.axis_index("c")
    b = c * (B // NC) + s // 2
    half = s % 2
    row0 = half * HROWS
    orow = c * NS + s
    sems = (sem0, sem1)

    def _copies1(g, p, sem):
        r0 = row0 + g * RPC
        cps = [pltpu.make_async_copy(
            lbl_hbm.at[b, 0, pl.ds(r0, RPC), :],
            lblb.at[pl.ds(p * RPC, RPC), :], sem)]
        for ch in range(C):
            cps.append(pltpu.make_async_copy(
                is_hbm.at[b, ch, pl.ds(r0, RPC), :],
                chb.at[pl.ds((p * C + ch) * RPC, RPC), :], sem))
        return cps

    def _copies2(g, p, sem):
        return _copies1(g, p, sem) + [pltpu.make_async_copy(
            il_hbm.at[b, pl.ds(row0 + g * RPC, RPC), :],
            ilb.at[pl.ds(p * RPC, RPC), :], sem)]

    zero = jnp.zeros((L,), jnp.float32)

    pltpu.sync_copy(th_hbm, thb)
    tv = thb[...]

    @plsc.parallel_loop(0, TAB // L, unroll=8)
    def _z(j):
        tab[pl.ds(j * L, L)] = zero

    iotav = _iota16()
    ones = jnp.full((L,), 1.0, jnp.float32)

    for p in range(2):
        for cp in _copies1(p, p, sems[p]):
            cp.start()

    @pl.loop(0, NCHUNK, step=2)
    def _pair1(g):
        for p in range(2):
            gg = g + p
            for cp in _copies1(gg, p, sems[p]):
                cp.wait()

            @plsc.parallel_loop(0, VPR, unroll=4)
            def _vreg(k):
                for r in range(RPC):
                    lbl = lblb[p * RPC + r, pl.ds(k * L, L)]
                    idx0 = (lbl << 4) + iotav
                    for ch in range(C):
                        v = chb[(p * C + ch) * RPC + r, pl.ds(k * L, L)]
                        plsc.addupdate_scatter(
                            tab, [idx0 + ch * (NSEG * L)], v)
                    plsc.addupdate_scatter(tab, [idx0 + C * (NSEG * L)],
                                           ones)

            @pl.when(gg + 2 < NCHUNK)
            def _():
                for cp in _copies1(gg + 2, p, sems[p]):
                    cp.start()

    last = iotav == (L - 1)

    @plsc.parallel_loop(0, LANE_TAB, unroll=4)
    def _red(e):
        cs = plsc.cumsum(tab[pl.ds(e * L, L)])
        plsc.store_scatter(comb, [iotav + (e - (L - 1))], cs, mask=last)

    pltpu.sync_copy(comb, shr.at[s])
    plsc.subcore_barrier()
    pltpu.sync_copy(shr.at[s ^ 1], part)

    @plsc.parallel_loop(0, LANE_TAB // L, unroll=2)
    def _add(j):
        o = j * L
        comb[pl.ds(o, L)] = comb[pl.ds(o, L)] + part[pl.ds(o, L)]

    def _bf16(a):
        u = plsc.bitcast(a, jnp.uint32)
        return (u + jnp.uint32(0x7FFF) + ((u >> 16) & jnp.uint32(1))) >> 16

    @plsc.parallel_loop(0, NSEG // L, unroll=2)
    def _avg(v):
        g0 = v * L
        n = comb[pl.ds(C * NSEG + g0, L)]
        inv = 1.0 / jnp.maximum(n, 1.0)
        keep = (g0 + _iota16()) != 0
        r = []
        for ch in range(C):
            a = comb[pl.ds(ch * NSEG + g0, L)] * inv
            a = jnp.where(keep, a, 0.0)
            r.append(_bf16(a))
        pckb[pl.ds(g0, L)] = plsc.bitcast((r[0] << 16) | r[1], jnp.int32)
        pckb[pl.ds(NSEG + g0, L)] = plsc.bitcast((r[2] << 16) | r[3],
                                                 jnp.int32)

    for p in range(2):
        for cp in _copies2(p, p, sems[p]):
            cp.start()

    def _pair2(g, acc):
        for p in range(2):
            gg = g * 2 + p
            for cp in _copies2(gg, p, sems[p]):
                cp.wait()

            @plsc.parallel_loop(0, VPR, unroll=2, carry=acc)
            def _vreg(k, a):
                hi = jnp.uint32(0xFFFF0000)
                for r in range(RPC):
                    lbl = lblb[p * RPC + r, pl.ds(k * L, L)]
                    il = ilb[p * RPC + r, pl.ds(k * L, L)]
                    q0 = plsc.bitcast(
                        plsc.load_gather(pckb, [lbl]), jnp.uint32)
                    q1 = plsc.bitcast(
                        plsc.load_gather(pckb, [lbl + NSEG]), jnp.uint32)
                    av = (plsc.bitcast(q0 & hi, jnp.float32),
                          plsc.bitcast(q0 << 16, jnp.float32),
                          plsc.bitcast(q1 & hi, jnp.float32),
                          plsc.bitcast(q1 << 16, jnp.float32))
                    nrm = zero
                    for ch in range(C):
                        v = chb[(p * C + ch) * RPC + r, pl.ds(k * L, L)]
                        d = v - av[ch]
                        nrm = nrm + d * d
                    w = jnp.where(il.astype(jnp.float32) > tv, 1.0, 0.0)
                    a = a + w * nrm
                return a
            acc = _vreg

            @pl.when(gg + 2 < NCHUNK)
            def _():
                for cp in _copies2(gg + 2, p, sems[p]):
                    cp.start()
        return acc

    acc = lax.fori_loop(0, NCHUNK // 2, _pair2, zero)
    accb[...] = acc
    pltpu.sync_copy(accb, out_hbm.at[orow])


def kernel(Is, Ispp, Il, line_thresh):
    th = jnp.full((L,), line_thresh, jnp.float32)
    parts = _superpixel(Is, Ispp, Il, th)
    return jnp.sum(parts) / (B * HW)
